# Initial kernel scaffold; baseline (speedup 1.0000x reference)
import jax
import jax.numpy as jnp
from jax.experimental import pallas as pl


def kernel(words, table):
    raise NotImplementedError("write your pallas kernel here")


# speedup vs baseline: 4.0951x; 4.0951x over previous
import functools

import jax
import jax.numpy as jnp
from jax import lax
from jax.experimental import pallas as pl
from jax.experimental.pallas import tpu as pltpu
from jax.experimental.pallas import tpu_sc as plsc

_STEP = 128
_LANES = 16


def _body(n_steps, nc, table_hbm, words_hbm, out_hbm, idx_v, rows_v, sem):
    wid = lax.axis_index("s") * nc + lax.axis_index("c")
    pltpu.sync_copy(words_hbm.at[wid], idx_v)

    def step(j, carry):
        pltpu.async_copy(table_hbm.at[idx_v.at[j]], rows_v, sem).wait()

        acc = jnp.zeros((_LANES,), jnp.int32)
        for g in range(_STEP // _LANES):
            v = idx_v[j, pl.ds(g * _LANES, _LANES)]
            acc = acc + (v == 0).astype(jnp.int32)
        zc = jnp.sum(acc)

        @pl.when(zc > 0)
        def _fixup():
            zero = jnp.zeros((_LANES,), jnp.float32)
            for g in range(_STEP // _LANES):
                v = idx_v[j, pl.ds(g * _LANES, _LANES)]
                m = v == 0
                rid = lax.iota(jnp.int32, _LANES) + g * _LANES
                for c in range(rows_v.shape[1]):
                    cid = jnp.full((_LANES,), c, jnp.int32)
                    plsc.store_scatter(rows_v, [rid, cid], zero, mask=m)


        pltpu.sync_copy(
            rows_v, out_hbm.at[pl.ds((wid * n_steps + j) * _STEP, _STEP)]
        )
        return carry

    lax.fori_loop(0, n_steps, step, 0)


def kernel(words, table):
    B, H = words.shape
    V, D = table.shape
    info = plsc.get_sparse_core_info()
    nc, ns = info.num_cores, info.num_subcores
    nw = nc * ns
    tot = B * H
    n_steps = tot // (nw * _STEP)
    words3 = words.reshape(nw, n_steps, _STEP).astype(jnp.int32)

    mesh = plsc.VectorSubcoreMesh(core_axis_name="c", subcore_axis_name="s")
    run = pl.kernel(
        functools.partial(_body, n_steps, nc),
        out_type=jax.ShapeDtypeStruct((tot, D), jnp.float32),
        mesh=mesh,
        compiler_params=pltpu.CompilerParams(use_tc_tiling_on_sc=False, needs_layout_passes=False),
        scratch_types=[
            pltpu.VMEM((n_steps, _STEP), jnp.int32),
            pltpu.VMEM((_STEP, D), jnp.float32),
            pltpu.SemaphoreType.DMA,
        ],
    )
    out = run(table, words3)
    return out.reshape(B, H, D)


# speedup vs baseline: 4.7186x; 1.1523x over previous
import functools

import jax
import jax.numpy as jnp
from jax import lax
from jax.experimental import pallas as pl
from jax.experimental.pallas import tpu as pltpu
from jax.experimental.pallas import tpu_sc as plsc

_STEP = 128
_LANES = 16
_NBUF = 8
_PREF = 4


def _body(n_steps, nc, table_hbm, words_hbm, out_hbm, idx_v, rows_v, gsem, ssem):
    wid = lax.axis_index("s") * nc + lax.axis_index("c")
    out_base = wid * n_steps * _STEP
    pltpu.sync_copy(words_hbm.at[wid], idx_v)

    def gather(j, slot):
        return pltpu.make_async_copy(
            table_hbm.at[idx_v.at[j]],
            rows_v.at[pl.ds(slot * _STEP, _STEP)],
            gsem.at[slot],
        )

    def store(j, slot):
        return pltpu.make_async_copy(
            rows_v.at[pl.ds(slot * _STEP, _STEP)],
            out_hbm.at[pl.ds(out_base + j * _STEP, _STEP)],
            ssem.at[slot],
        )

    for j in range(_PREF):
        gather(j, j % _NBUF).start()

    def step(j, carry):
        slot = lax.rem(j, _NBUF)

        @pl.when(j >= _PREF)
        def _():
            store(j - _PREF, lax.rem(j - _PREF, _NBUF)).wait()

        @pl.when(j + _PREF < n_steps)
        def _():
            gather(j + _PREF, lax.rem(j + _PREF, _NBUF)).start()

        gather(j, slot).wait()

        acc = jnp.zeros((_LANES,), jnp.int32)
        for g in range(_STEP // _LANES):
            v = idx_v[j, pl.ds(g * _LANES, _LANES)]
            acc = acc + (v == 0).astype(jnp.int32)
        zc = jnp.sum(acc)

        @pl.when(zc > 0)
        def _fixup():
            zero = jnp.zeros((_LANES,), jnp.float32)
            for g in range(_STEP // _LANES):
                v = idx_v[j, pl.ds(g * _LANES, _LANES)]
                m = v == 0
                rid = lax.iota(jnp.int32, _LANES) + slot * _STEP + g * _LANES
                for c in range(rows_v.shape[1]):
                    cid = jnp.full((_LANES,), c, jnp.int32)
                    plsc.store_scatter(rows_v, [rid, cid], zero, mask=m)

        store(j, slot).start()
        return carry

    lax.fori_loop(0, n_steps, step, 0)

    for j in range(n_steps - _PREF, n_steps):
        store(j, j % _NBUF).wait()


def kernel(words, table):
    B, H = words.shape
    V, D = table.shape
    info = plsc.get_sparse_core_info()
    nc, ns = info.num_cores, info.num_subcores
    nw = nc * ns
    tot = B * H
    n_steps = tot // (nw * _STEP)
    words3 = words.reshape(nw, n_steps, _STEP).astype(jnp.int32)

    mesh = plsc.VectorSubcoreMesh(core_axis_name="c", subcore_axis_name="s")
    run = pl.kernel(
        functools.partial(_body, n_steps, nc),
        out_type=jax.ShapeDtypeStruct((tot, D), jnp.float32),
        mesh=mesh,
        compiler_params=pltpu.CompilerParams(
            use_tc_tiling_on_sc=False, needs_layout_passes=False
        ),
        scratch_types=[
            pltpu.VMEM((n_steps, _STEP), jnp.int32),
            pltpu.VMEM((_NBUF * _STEP, D), jnp.float32),
            pltpu.SemaphoreType.DMA((_NBUF,)),
            pltpu.SemaphoreType.DMA((_NBUF,)),
        ],
    )
    out = run(table, words3)
    return out.reshape(B, H, D)
